# baseline (speedup 1.0000x reference)
import functools
import jax
import jax.numpy as jnp
from jax import lax
from jax.experimental import pallas as pl
from jax.experimental.pallas import tpu as pltpu
from jax.experimental.pallas import tpu_sc as plsc

BATCH = 16384
EMB_DIM = 32
NUM_CORES = 2
NUM_SUBCORES = 16
NUM_WORKERS = NUM_CORES * NUM_SUBCORES
B_PER_W = BATCH // NUM_WORKERS
LANES = 16


def _dot_kernel(user_hbm, item_hbm, uemb_hbm, iemb_hbm, out_hbm,
                uidx_v, iidx_v, urows_v, irows_v, out_v, sem_u, sem_i):
    wid = lax.axis_index("s") * NUM_CORES + lax.axis_index("c")
    base = wid * B_PER_W

    pltpu.sync_copy(user_hbm.at[pl.ds(base, B_PER_W)], uidx_v)
    pltpu.sync_copy(item_hbm.at[pl.ds(base, B_PER_W)], iidx_v)

    cp_u = pltpu.async_copy(uemb_hbm.at[uidx_v], urows_v, sem_u)
    cp_i = pltpu.async_copy(iemb_hbm.at[iidx_v], irows_v, sem_i)
    cp_u.wait()
    cp_i.wait()

    iota16 = lax.iota(jnp.int32, LANES)

    @pl.loop(0, B_PER_W // LANES)
    def _(g):
        row_ids = jnp.full((LANES,), g * LANES, jnp.int32) + iota16
        acc = jnp.zeros((LANES,), jnp.float32)
        for d in range(EMB_DIM):
            col = jnp.full((LANES,), d, jnp.int32)
            u = plsc.load_gather(urows_v, [row_ids, col])
            v = plsc.load_gather(irows_v, [row_ids, col])
            acc = acc + u * v
        out_v[pl.ds(g * LANES, LANES)] = acc

    pltpu.sync_copy(out_v, out_hbm.at[pl.ds(base, B_PER_W)])


@jax.jit
def kernel(user, item, user_emb, item_emb):
    mesh = plsc.VectorSubcoreMesh(core_axis_name="c", subcore_axis_name="s")
    run = pl.kernel(
        _dot_kernel,
        out_type=jax.ShapeDtypeStruct((BATCH,), jnp.float32),
        mesh=mesh,
        compiler_params=pltpu.CompilerParams(needs_layout_passes=False,
                                             use_tc_tiling_on_sc=False),
        scratch_types=[
            pltpu.VMEM((B_PER_W,), jnp.int32),
            pltpu.VMEM((B_PER_W,), jnp.int32),
            pltpu.VMEM((B_PER_W, EMB_DIM), jnp.float32),
            pltpu.VMEM((B_PER_W, EMB_DIM), jnp.float32),
            pltpu.VMEM((B_PER_W,), jnp.float32),
            pltpu.SemaphoreType.DMA,
            pltpu.SemaphoreType.DMA,
        ],
    )
    return run(user.astype(jnp.int32), item.astype(jnp.int32),
               user_emb, item_emb)
